# Initial kernel scaffold; baseline (speedup 1.0000x reference)
import jax
import jax.numpy as jnp
from jax.experimental import pallas as pl


def kernel(pos, W_now, W_next):
    raise NotImplementedError("write your pallas kernel here")


# speedup vs baseline: 4.9301x; 4.9301x over previous
import functools

import jax
import jax.numpy as jnp
from jax import lax
from jax.experimental import pallas as pl
from jax.experimental.pallas import tpu as pltpu, tpu_sc as plsc

_VOCAB = 10000
_HALF = 64
_NC, _NS = 2, 16
_NW = _NC * _NS
_IPS = 128
_J = 8
_K = _IPS * _J


@functools.partial(jax.jit, static_argnames=("total_rows",))
def _sc_gather(table, idx2d, total_rows):
    rows_per_worker = total_rows // _NW
    chunks = rows_per_worker // _K
    mesh = plsc.VectorSubcoreMesh(core_axis_name="c", subcore_axis_name="s")

    @functools.partial(
        pl.kernel,
        out_type=jax.ShapeDtypeStruct((total_rows, _HALF), jnp.float32),
        mesh=mesh,
        scratch_types=[
            pltpu.VMEM((_J, _IPS), jnp.int32),
            pltpu.VMEM((_K, _HALF), jnp.float32),
            pltpu.SemaphoreType.DMA,
        ],
        compiler_params=pltpu.CompilerParams(use_tc_tiling_on_sc=False),
    )
    def k(table_hbm, idx_hbm, out_hbm, idx_v, rows_v, sem):
        wid = lax.axis_index("s") * _NC + lax.axis_index("c")
        base = wid * rows_per_worker

        @pl.loop(0, chunks)
        def _chunk(g):
            row0 = pl.multiple_of(base + g * _K, _K)
            pltpu.sync_copy(
                idx_hbm.at[pl.ds(pl.multiple_of(row0 // _IPS, _J), _J)], idx_v
            )
            descs = [
                pltpu.async_copy(
                    table_hbm.at[idx_v.at[j]],
                    rows_v.at[pl.ds(j * _IPS, _IPS)],
                    sem,
                )
                for j in range(_J)
            ]
            for d in descs:
                d.wait()
            pltpu.sync_copy(rows_v, out_hbm.at[pl.ds(row0, _K)])

    return k(table, idx2d)


def kernel(pos, W_now, W_next):
    B, _, L = pos.shape
    table = jnp.concatenate([W_now, W_next], axis=0)
    offs = jnp.array([0, _VOCAB], dtype=pos.dtype).reshape(1, 2, 1)
    idx = (pos + offs).transpose(0, 2, 1).reshape(-1).astype(jnp.int32)
    total_rows = idx.shape[0]
    idx2d = idx.reshape(total_rows // _IPS, _IPS)
    out = _sc_gather(table, idx2d, total_rows)
    return out.reshape(B, L, 2 * _HALF)


# speedup vs baseline: 5.0725x; 1.0289x over previous
import functools

import jax
import jax.numpy as jnp
from jax import lax
from jax.experimental import pallas as pl
from jax.experimental.pallas import tpu as pltpu, tpu_sc as plsc

_VOCAB = 10000
_HALF = 64
_NC, _NS = 2, 16
_NW = _NC * _NS
_IPS = 128
_J = 4
_K = _IPS * _J
_IB = 8


@functools.partial(jax.jit, static_argnames=("total_rows",))
def _sc_gather(table, idx2d, total_rows):
    rows_per_worker = total_rows // _NW
    chunks = rows_per_worker // _K
    blocks = chunks // 2
    mesh = plsc.VectorSubcoreMesh(core_axis_name="c", subcore_axis_name="s")

    @functools.partial(
        pl.kernel,
        out_type=jax.ShapeDtypeStruct((total_rows, _HALF), jnp.float32),
        mesh=mesh,
        scratch_types=[
            pltpu.VMEM((2, _IB, _IPS), jnp.int32),
            pltpu.VMEM((2, _K, _HALF), jnp.float32),
            pltpu.SemaphoreType.DMA,
            pltpu.SemaphoreType.DMA,
            pltpu.SemaphoreType.DMA,
            pltpu.SemaphoreType.DMA,
            pltpu.SemaphoreType.DMA,
            pltpu.SemaphoreType.DMA,
        ],
        compiler_params=pltpu.CompilerParams(use_tc_tiling_on_sc=False),
    )
    def k(table_hbm, idx_hbm, out_hbm, idx_v, rows_v,
          isem0, isem1, gsem0, gsem1, osem0, osem1):
        isems, gsems, osems = (isem0, isem1), (gsem0, gsem1), (osem0, osem1)
        wid = lax.axis_index("s") * _NC + lax.axis_index("c")
        chunk0 = wid * chunks
        block0 = wid * blocks

        def idx_load(blk, t):
            return pltpu.make_async_copy(
                idx_hbm.at[pl.ds((block0 + blk) * _IB, _IB)], idx_v.at[t],
                isems[t])

        def out_store(c, rs):
            return pltpu.make_async_copy(
                rows_v.at[rs], out_hbm.at[pl.ds((chunk0 + c) * _K, _K)],
                osems[rs])

        idx_load(0, 0).start()
        idx_load(1, 1).start()

        @pl.loop(0, chunks, step=4)
        def _body(g):
            blk = g // 2
            for s in range(4):
                c = g + s
                rs = s % 2
                t = s // 2
                half = s % 2
                if s in (0, 2):
                    idx_load(blk + t, t).wait()

                @pl.when(c >= 2)
                def _():
                    out_store(c - 2, rs).wait()

                descs = [
                    pltpu.async_copy(
                        table_hbm.at[idx_v.at[t, half * _J + j]],
                        rows_v.at[rs, pl.ds(j * _IPS, _IPS)],
                        gsems[rs],
                    )
                    for j in range(_J)
                ]
                for d in descs:
                    d.wait()

                if s in (1, 3):
                    @pl.when(blk + t + 2 < blocks)
                    def _():
                        idx_load(blk + t + 2, t).start()

                out_store(c, rs).start()

        out_store(chunks - 2, 0).wait()
        out_store(chunks - 1, 1).wait()

    return k(table, idx2d)


def kernel(pos, W_now, W_next):
    B, _, L = pos.shape
    table = jnp.concatenate([W_now, W_next], axis=0)
    offs = jnp.array([0, _VOCAB], dtype=pos.dtype).reshape(1, 2, 1)
    idx = (pos + offs).transpose(0, 2, 1).reshape(-1).astype(jnp.int32)
    total_rows = idx.shape[0]
    idx2d = idx.reshape(total_rows // _IPS, _IPS)
    out = _sc_gather(table, idx2d, total_rows)
    return out.reshape(B, L, 2 * _HALF)


# speedup vs baseline: 5.6345x; 1.1108x over previous
import functools

import jax
import jax.numpy as jnp
from jax import lax
from jax.experimental import pallas as pl
from jax.experimental.pallas import tpu as pltpu, tpu_sc as plsc

_VOCAB = 10000
_HALF = 64
_NC, _NS = 2, 16
_NW = _NC * _NS
_IPS = 128
_J = 2
_K = _IPS * _J
_IB = 8
_CPB = _IB * _IPS // _K


@functools.partial(jax.jit, static_argnames=("total_rows",))
def _sc_gather(table, idx2d, total_rows):
    rows_per_worker = total_rows // _NW
    chunks = rows_per_worker // _K
    blocks = chunks // _CPB
    mesh = plsc.VectorSubcoreMesh(core_axis_name="c", subcore_axis_name="s")

    @functools.partial(
        pl.kernel,
        out_type=jax.ShapeDtypeStruct((total_rows, _HALF), jnp.float32),
        mesh=mesh,
        scratch_types=[
            pltpu.VMEM((2, _IB, _IPS), jnp.int32),
            pltpu.VMEM((2, _K, _HALF), jnp.float32),
            pltpu.VMEM_SHARED((2 * _VOCAB, _HALF), jnp.float32),
            pltpu.SemaphoreType.DMA,
            pltpu.SemaphoreType.DMA,
            pltpu.SemaphoreType.DMA,
            pltpu.SemaphoreType.DMA,
            pltpu.SemaphoreType.DMA,
            pltpu.SemaphoreType.DMA,
        ],
        compiler_params=pltpu.CompilerParams(use_tc_tiling_on_sc=False),
    )
    def k(table_hbm, idx_hbm, out_hbm, idx_v, rows_v, table_sp,
          isem0, isem1, gsem0, gsem1, osem0, osem1):
        isems, gsems, osems = (isem0, isem1), (gsem0, gsem1), (osem0, osem1)
        wid = lax.axis_index("s") * _NC + lax.axis_index("c")
        chunk0 = wid * chunks
        block0 = wid * blocks

        sid = lax.axis_index("s")
        stripe = 2 * _VOCAB // _NS
        pltpu.sync_copy(
            table_hbm.at[pl.ds(sid * stripe, stripe)],
            table_sp.at[pl.ds(sid * stripe, stripe)],
        )
        plsc.subcore_barrier()

        def idx_load(blk, t):
            return pltpu.make_async_copy(
                idx_hbm.at[pl.ds((block0 + blk) * _IB, _IB)], idx_v.at[t],
                isems[t])

        def out_store(c, rs):
            return pltpu.make_async_copy(
                rows_v.at[rs], out_hbm.at[pl.ds((chunk0 + c) * _K, _K)],
                osems[rs])

        idx_load(0, 0).start()
        idx_load(1, 1).start()

        @pl.loop(0, chunks, step=2 * _CPB)
        def _body(g):
            blk = g // _CPB
            for s in range(2 * _CPB):
                c = g + s
                rs = s % 2
                t = s // _CPB
                part = s % _CPB
                if part == 0:
                    idx_load(blk + t, t).wait()

                @pl.when(c >= 2)
                def _():
                    out_store(c - 2, rs).wait()

                descs = [
                    pltpu.async_copy(
                        table_sp.at[idx_v.at[t, part * _J + j]],
                        rows_v.at[rs, pl.ds(j * _IPS, _IPS)],
                        gsems[rs],
                    )
                    for j in range(_J)
                ]
                for d in descs:
                    d.wait()

                if part == _CPB - 1:
                    @pl.when(blk + t + 2 < blocks)
                    def _():
                        idx_load(blk + t + 2, t).start()

                out_store(c, rs).start()

        out_store(chunks - 2, 0).wait()
        out_store(chunks - 1, 1).wait()

    return k(table, idx2d)


def kernel(pos, W_now, W_next):
    B, _, L = pos.shape
    table = jnp.concatenate([W_now, W_next], axis=0)
    offs = jnp.array([0, _VOCAB], dtype=pos.dtype).reshape(1, 2, 1)
    idx = (pos + offs).transpose(0, 2, 1).reshape(-1).astype(jnp.int32)
    total_rows = idx.shape[0]
    idx2d = idx.reshape(total_rows // _IPS, _IPS)
    out = _sc_gather(table, idx2d, total_rows)
    return out.reshape(B, L, 2 * _HALF)


# speedup vs baseline: 31.5445x; 5.5985x over previous
import functools

import jax
import jax.numpy as jnp
from jax import lax
from jax.experimental import pallas as pl
from jax.experimental.pallas import tpu as pltpu, tpu_sc as plsc

_VOCAB = 10000
_HALF = 64
_NC, _NS = 2, 16
_NW = _NC * _NS
_PARTS = ((0, 72), (72, 64), (136, 64))
_PMAX = 72


@functools.partial(jax.jit, static_argnames=("B", "L"))
def _sc_embed(pos, W_now, W_next, B, L):
    b_per_w = B // _NW
    nparts = len(_PARTS)
    mesh = plsc.VectorSubcoreMesh(core_axis_name="c", subcore_axis_name="s")

    @functools.partial(
        pl.kernel,
        out_type=jax.ShapeDtypeStruct((B * L, 2, _HALF), jnp.float32),
        mesh=mesh,
        scratch_types=[
            pltpu.VMEM((2, 2, L), jnp.int32),
            pltpu.VMEM((2, _PMAX, _HALF), jnp.float32),
            pltpu.VMEM((2, _PMAX, _HALF), jnp.float32),
            pltpu.VMEM_SHARED((_VOCAB, _HALF), jnp.float32),
            pltpu.VMEM_SHARED((_VOCAB, _HALF), jnp.float32),
            pltpu.SemaphoreType.DMA,
            pltpu.SemaphoreType.DMA,
            pltpu.SemaphoreType.DMA,
            pltpu.SemaphoreType.DMA,
            pltpu.SemaphoreType.DMA,
            pltpu.SemaphoreType.DMA,
            pltpu.SemaphoreType.DMA,
            pltpu.SemaphoreType.DMA,
        ],
        compiler_params=pltpu.CompilerParams(use_tc_tiling_on_sc=False),
    )
    def k(pos_hbm, wn_hbm, wx_hbm, out_hbm, idx_v, bn_v, bx_v, wn_sp, wx_sp,
          isem0, isem1, gsem0, gsem1, on0, on1, ox0, ox1):
        isems, gsems = (isem0, isem1), (gsem0, gsem1)
        osems_n, osems_x = (on0, on1), (ox0, ox1)
        wid = lax.axis_index("s") * _NC + lax.axis_index("c")
        b0 = wid * b_per_w

        sid = lax.axis_index("s")
        stripe = _VOCAB // _NS
        pltpu.sync_copy(wn_hbm.at[pl.ds(sid * stripe, stripe)],
                        wn_sp.at[pl.ds(sid * stripe, stripe)])
        pltpu.sync_copy(wx_hbm.at[pl.ds(sid * stripe, stripe)],
                        wx_sp.at[pl.ds(sid * stripe, stripe)])
        plsc.subcore_barrier()

        def idx_load(b, t):
            return pltpu.make_async_copy(pos_hbm.at[b0 + b], idx_v.at[t],
                                         isems[t])

        def store_now(b, part, s):
            off, ln = _PARTS[part]
            return pltpu.make_async_copy(
                bn_v.at[s, pl.ds(0, ln)],
                out_hbm.at[pl.ds((b0 + b) * L + off, ln), 0], osems_n[s])

        def store_next(b, part, s):
            off, ln = _PARTS[part]
            return pltpu.make_async_copy(
                bx_v.at[s, pl.ds(0, ln)],
                out_hbm.at[pl.ds((b0 + b) * L + off, ln), 1], osems_x[s])

        idx_load(0, 0).start()
        idx_load(1, 1).start()

        @pl.loop(0, b_per_w, step=2)
        def _body(bb):
            for u in range(2):
                b = bb + u
                q0 = (bb + u) * nparts
                idx_load(b, u).wait()
                for part in range(nparts):
                    off, ln = _PARTS[part]
                    q = q0 + part
                    s = (u * nparts + part) % 2

                    part_prev = (part + 2 * nparts - 2) % nparts

                    @pl.when(q >= 2)
                    def _():
                        store_now(0, part_prev, s).wait()
                        store_next(0, part_prev, s).wait()

                    gn = pltpu.async_copy(
                        wn_sp.at[idx_v.at[u, 0, pl.ds(off, ln)]],
                        bn_v.at[s, pl.ds(0, ln)], gsems[s])
                    gx = pltpu.async_copy(
                        wx_sp.at[idx_v.at[u, 1, pl.ds(off, ln)]],
                        bx_v.at[s, pl.ds(0, ln)], gsems[s])
                    gn.wait()
                    gx.wait()

                    if part == nparts - 1:
                        @pl.when(b + 2 < b_per_w)
                        def _():
                            idx_load(b + 2, u).start()

                    store_now(b, part, s).start()
                    store_next(b, part, s).start()

        total_q = b_per_w * nparts
        for q in (total_q - 2, total_q - 1):
            store_now(0, q % nparts, q % 2).wait()
            store_next(0, q % nparts, q % 2).wait()

    return k(pos, W_now, W_next)


def kernel(pos, W_now, W_next):
    B, _, L = pos.shape
    out = _sc_embed(pos.astype(jnp.int32), W_now, W_next, B, L)
    return out.reshape(B, L, 2 * _HALF)


# speedup vs baseline: 32.7055x; 1.0368x over previous
import functools

import jax
import jax.numpy as jnp
from jax import lax
from jax.experimental import pallas as pl
from jax.experimental.pallas import tpu as pltpu, tpu_sc as plsc

_VOCAB = 10000
_HALF = 64
_NC, _NS = 2, 16
_NW = _NC * _NS
_PARTS = ((0, 96), (96, 104))
_PMAX = 104


@functools.partial(jax.jit, static_argnames=("B", "L"))
def _sc_embed(pos, W_now, W_next, B, L):
    b_per_w = B // _NW
    nparts = len(_PARTS)
    mesh = plsc.VectorSubcoreMesh(core_axis_name="c", subcore_axis_name="s")

    @functools.partial(
        pl.kernel,
        out_type=jax.ShapeDtypeStruct((B * L, 2, _HALF), jnp.float32),
        mesh=mesh,
        scratch_types=[
            pltpu.VMEM((2, 2, L), jnp.int32),
            pltpu.VMEM((2, _PMAX, _HALF), jnp.float32),
            pltpu.VMEM((2, _PMAX, _HALF), jnp.float32),
            pltpu.VMEM_SHARED((_VOCAB, _HALF), jnp.float32),
            pltpu.VMEM_SHARED((_VOCAB, _HALF), jnp.float32),
            pltpu.SemaphoreType.DMA,
            pltpu.SemaphoreType.DMA,
            pltpu.SemaphoreType.DMA,
            pltpu.SemaphoreType.DMA,
            pltpu.SemaphoreType.DMA,
            pltpu.SemaphoreType.DMA,
            pltpu.SemaphoreType.DMA,
            pltpu.SemaphoreType.DMA,
        ],
        compiler_params=pltpu.CompilerParams(use_tc_tiling_on_sc=False),
    )
    def k(pos_hbm, wn_hbm, wx_hbm, out_hbm, idx_v, bn_v, bx_v, wn_sp, wx_sp,
          isem0, isem1, gsem0, gsem1, on0, on1, ox0, ox1):
        isems, gsems = (isem0, isem1), (gsem0, gsem1)
        osems_n, osems_x = (on0, on1), (ox0, ox1)
        wid = lax.axis_index("s") * _NC + lax.axis_index("c")
        b0 = wid * b_per_w

        sid = lax.axis_index("s")
        stripe = _VOCAB // _NS
        pltpu.sync_copy(wn_hbm.at[pl.ds(sid * stripe, stripe)],
                        wn_sp.at[pl.ds(sid * stripe, stripe)])
        pltpu.sync_copy(wx_hbm.at[pl.ds(sid * stripe, stripe)],
                        wx_sp.at[pl.ds(sid * stripe, stripe)])
        plsc.subcore_barrier()

        def idx_load(b, t):
            return pltpu.make_async_copy(pos_hbm.at[b0 + b], idx_v.at[t],
                                         isems[t])

        def store_now(b, part, s):
            off, ln = _PARTS[part]
            return pltpu.make_async_copy(
                bn_v.at[s, pl.ds(0, ln)],
                out_hbm.at[pl.ds((b0 + b) * L + off, ln), 0], osems_n[s])

        def store_next(b, part, s):
            off, ln = _PARTS[part]
            return pltpu.make_async_copy(
                bx_v.at[s, pl.ds(0, ln)],
                out_hbm.at[pl.ds((b0 + b) * L + off, ln), 1], osems_x[s])

        idx_load(0, 0).start()
        idx_load(1, 1).start()

        @pl.loop(0, b_per_w, step=2)
        def _body(bb):
            for u in range(2):
                b = bb + u
                q0 = (bb + u) * nparts
                idx_load(b, u).wait()
                for part in range(nparts):
                    off, ln = _PARTS[part]
                    q = q0 + part
                    s = (u * nparts + part) % 2

                    part_prev = (part + 2 * nparts - 2) % nparts

                    @pl.when(q >= 2)
                    def _():
                        store_now(0, part_prev, s).wait()
                        store_next(0, part_prev, s).wait()

                    gn = pltpu.async_copy(
                        wn_sp.at[idx_v.at[u, 0, pl.ds(off, ln)]],
                        bn_v.at[s, pl.ds(0, ln)], gsems[s])
                    gx = pltpu.async_copy(
                        wx_sp.at[idx_v.at[u, 1, pl.ds(off, ln)]],
                        bx_v.at[s, pl.ds(0, ln)], gsems[s])
                    gn.wait()
                    gx.wait()

                    if part == nparts - 1:
                        @pl.when(b + 2 < b_per_w)
                        def _():
                            idx_load(b + 2, u).start()

                    store_now(b, part, s).start()
                    store_next(b, part, s).start()

        total_q = b_per_w * nparts
        for q in (total_q - 2, total_q - 1):
            store_now(0, q % nparts, q % 2).wait()
            store_next(0, q % nparts, q % 2).wait()

    return k(pos, W_now, W_next)


def kernel(pos, W_now, W_next):
    B, _, L = pos.shape
    out = _sc_embed(pos.astype(jnp.int32), W_now, W_next, B, L)
    return out.reshape(B, L, 2 * _HALF)


# speedup vs baseline: 34.3714x; 1.0509x over previous
import functools

import jax
import jax.numpy as jnp
from jax import lax
from jax.experimental import pallas as pl
from jax.experimental.pallas import tpu as pltpu, tpu_sc as plsc

_VOCAB = 10000
_HALF = 64
_NC, _NS = 2, 16
_NW = _NC * _NS
_PARTS = ((0, 96), (96, 104))
_PMAX = 104
_NP = len(_PARTS)


@functools.partial(jax.jit, static_argnames=("B", "L"))
def _sc_embed(pos, W_now, W_next, B, L):
    b_per_w = B // _NW
    total_q = b_per_w * _NP
    mesh = plsc.VectorSubcoreMesh(core_axis_name="c", subcore_axis_name="s")

    @functools.partial(
        pl.kernel,
        out_type=jax.ShapeDtypeStruct((B * L, 2, _HALF), jnp.float32),
        mesh=mesh,
        scratch_types=[
            pltpu.VMEM((2, 2, L), jnp.int32),
            pltpu.VMEM((2, _PMAX, _HALF), jnp.float32),
            pltpu.VMEM((2, _PMAX, _HALF), jnp.float32),
            pltpu.VMEM_SHARED((_VOCAB, _HALF), jnp.float32),
            pltpu.VMEM_SHARED((_VOCAB, _HALF), jnp.float32),
            pltpu.SemaphoreType.DMA,
            pltpu.SemaphoreType.DMA,
            pltpu.SemaphoreType.DMA,
            pltpu.SemaphoreType.DMA,
            pltpu.SemaphoreType.DMA,
            pltpu.SemaphoreType.DMA,
            pltpu.SemaphoreType.DMA,
            pltpu.SemaphoreType.DMA,
        ],
        compiler_params=pltpu.CompilerParams(use_tc_tiling_on_sc=False),
    )
    def k(pos_hbm, wn_hbm, wx_hbm, out_hbm, idx_v, bn_v, bx_v, wn_sp, wx_sp,
          isem0, isem1, gsem0, gsem1, on0, on1, ox0, ox1):
        isems, gsems = (isem0, isem1), (gsem0, gsem1)
        osems_n, osems_x = (on0, on1), (ox0, ox1)
        wid = lax.axis_index("s") * _NC + lax.axis_index("c")
        b0 = wid * b_per_w

        sid = lax.axis_index("s")
        stripe = _VOCAB // _NS
        pltpu.sync_copy(wn_hbm.at[pl.ds(sid * stripe, stripe)],
                        wn_sp.at[pl.ds(sid * stripe, stripe)])
        pltpu.sync_copy(wx_hbm.at[pl.ds(sid * stripe, stripe)],
                        wx_sp.at[pl.ds(sid * stripe, stripe)])
        plsc.subcore_barrier()

        def idx_load(b, t):
            return pltpu.make_async_copy(pos_hbm.at[b0 + b], idx_v.at[t],
                                         isems[t])

        def gathers_start(b_t, part, s):
            off, ln = _PARTS[part]
            pltpu.async_copy(wn_sp.at[idx_v.at[b_t, 0, pl.ds(off, ln)]],
                             bn_v.at[s, pl.ds(0, ln)], gsems[s])
            pltpu.async_copy(wx_sp.at[idx_v.at[b_t, 1, pl.ds(off, ln)]],
                             bx_v.at[s, pl.ds(0, ln)], gsems[s])

        def gathers_wait(part, s):
            _, ln = _PARTS[part]
            pltpu.make_async_copy(wn_hbm.at[pl.ds(0, ln)],
                                  bn_v.at[s, pl.ds(0, ln)], gsems[s]).wait()
            pltpu.make_async_copy(wx_hbm.at[pl.ds(0, ln)],
                                  bx_v.at[s, pl.ds(0, ln)], gsems[s]).wait()

        def store_now(b, part, s):
            off, ln = _PARTS[part]
            return pltpu.make_async_copy(
                bn_v.at[s, pl.ds(0, ln)],
                out_hbm.at[pl.ds((b0 + b) * L + off, ln), 0], osems_n[s])

        def store_next(b, part, s):
            off, ln = _PARTS[part]
            return pltpu.make_async_copy(
                bx_v.at[s, pl.ds(0, ln)],
                out_hbm.at[pl.ds((b0 + b) * L + off, ln), 1], osems_x[s])

        idx_load(0, 0).start()
        idx_load(1, 1).start()

        @pl.loop(0, total_q, step=2 * _NP)
        def _body(g):
            for u in range(2 * _NP):
                q = g + u
                part = u % _NP
                s = u % 2
                b = q // _NP
                b_t = (u // _NP) % 2

                if part == 0:
                    idx_load(0, b_t).wait()

                @pl.when(q >= 2)
                def _():
                    store_now(0, part, s).wait()
                    store_next(0, part, s).wait()

                gathers_start(b_t, part, s)

                part_m1 = (u - 1) % _NP
                s_m1 = 1 - s
                b_t_m1 = ((u - 1) % (2 * _NP)) // _NP

                @pl.when(q >= 1)
                def _():
                    b_m1 = jnp.maximum(q - 1, 0) // _NP
                    gathers_wait(part_m1, s_m1)
                    store_now(b_m1, part_m1, s_m1).start()
                    store_next(b_m1, part_m1, s_m1).start()

                if part_m1 == _NP - 1:
                    @pl.when((q >= 1) & (q - 1 + 2 * _NP < total_q))
                    def _():
                        idx_load(jnp.maximum(q - 1, 0) // _NP + 2, b_t_m1).start()

        q_last = total_q - 1
        gathers_wait(q_last % _NP, q_last % 2)
        store_now(b_per_w - 1, q_last % _NP, q_last % 2).start()
        store_next(b_per_w - 1, q_last % _NP, q_last % 2).start()
        for q in (total_q - 2, total_q - 1):
            store_now(0, q % _NP, q % 2).wait()
            store_next(0, q % _NP, q % 2).wait()

    return k(pos, W_now, W_next)


def kernel(pos, W_now, W_next):
    B, _, L = pos.shape
    out = _sc_embed(pos.astype(jnp.int32), W_now, W_next, B, L)
    return out.reshape(B, L, 2 * _HALF)
